# baseline (speedup 1.0000x reference)
import functools

import numpy as np
import jax
import jax.numpy as jnp
from jax import lax
from jax.experimental import pallas as pl
from jax.experimental.pallas import tpu as pltpu
from jax.experimental.pallas import tpu_sc as plsc

N = 10000
E = 160000
NB = 5
K = 10
C = 16
NUM_CLASSES = 6890
W128 = 128

NWORK = 32
NSLOT = 6
DEPTH = 3
NSLOT_S = 2
DEPTH_S = 1
CHUNK = 128
NCH_W = 40
E_PAD = NWORK * NCH_W * CHUNK
NJ = E_PAD // CHUNK
N_PAD = 10240
NPER = N_PAD // 16

TE = 1280
TN = 2000


def _sc_gather(table, idx2d):
    mesh = plsc.VectorSubcoreMesh(core_axis_name="c", subcore_axis_name="s")

    def body(table_hbm, idx_hbm, out_hbm, idx_v, buf, gsem, osem):
        wid = lax.axis_index("s") * 2 + lax.axis_index("c")
        base = wid * NCH_W
        pltpu.sync_copy(idx_hbm.at[pl.ds(base, NCH_W)], idx_v)
        for b in range(DEPTH):
            pltpu.async_copy(table_hbm.at[idx_v.at[b]], buf.at[b], gsem)

        def step(j, carry):
            slot = lax.rem(j, NSLOT)
            pltpu.make_async_copy(
                table_hbm.at[idx_v.at[j]], buf.at[slot], gsem).wait()
            pltpu.async_copy(buf.at[slot],
                             out_hbm.at[pl.ds((base + j) * CHUNK, CHUNK)],
                             osem)

            @pl.when(j + DEPTH < NCH_W)
            def _():
                @pl.when(j >= DEPTH)
                def _():
                    pltpu.make_async_copy(
                        buf.at[0], out_hbm.at[pl.ds(0, CHUNK)], osem).wait()
                pltpu.async_copy(
                    table_hbm.at[idx_v.at[j + DEPTH]],
                    buf.at[lax.rem(j + DEPTH, NSLOT)], gsem)

            return carry

        lax.fori_loop(0, NCH_W, step, 0)
        for _ in range(NSLOT):
            pltpu.make_async_copy(
                buf.at[0], out_hbm.at[pl.ds(0, CHUNK)], osem).wait()

    f = pl.kernel(
        body,
        out_type=jax.ShapeDtypeStruct((E_PAD, W128), jnp.float32),
        mesh=mesh,
        scratch_types=[
            pltpu.VMEM((NCH_W, CHUNK), jnp.int32),
            pltpu.VMEM((NSLOT, CHUNK, W128), jnp.float32),
            pltpu.SemaphoreType.DMA,
            pltpu.SemaphoreType.DMA,
        ],
    )
    return f(table, idx2d)


def _sc_scatter(msg, idx2d, zeros):
    mesh = plsc.VectorSubcoreMesh(core_axis_name="c", subcore_axis_name="s")

    def body(msg_hbm, idx_hbm, z_hbm, out_hbm, idx_v, buf, agg_sh, rsem,
             ssem):
        cc = lax.axis_index("c")
        ss = lax.axis_index("s")
        base = (ss * 2 + cc) * NCH_W
        pltpu.sync_copy(z_hbm.at[pl.ds(ss * NPER, NPER)],
                        agg_sh.at[pl.ds(ss * NPER, NPER)])
        pltpu.sync_copy(idx_hbm.at[pl.ds(base, NCH_W)], idx_v)
        plsc.subcore_barrier()
        for b in range(DEPTH_S):
            pltpu.async_copy(
                msg_hbm.at[pl.ds((base + b) * CHUNK, CHUNK)], buf.at[b], rsem)

        def step(j, carry):
            slot = lax.rem(j, NSLOT_S)
            pltpu.make_async_copy(
                msg_hbm.at[pl.ds((base + j) * CHUNK, CHUNK)], buf.at[slot],
                rsem).wait()
            pltpu.async_copy(buf.at[slot], agg_sh.at[idx_v.at[j]], ssem,
                             add=True)

            @pl.when(j + DEPTH_S < NCH_W)
            def _():
                @pl.when(j >= DEPTH_S)
                def _():
                    pltpu.make_async_copy(
                        buf.at[0], agg_sh.at[idx_v.at[0]], ssem).wait()
                pltpu.async_copy(
                    msg_hbm.at[pl.ds((base + j + DEPTH_S) * CHUNK, CHUNK)],
                    buf.at[lax.rem(j + DEPTH_S, NSLOT_S)], rsem)

            return carry

        lax.fori_loop(0, NCH_W, step, 0)
        for _ in range(NSLOT_S):
            pltpu.make_async_copy(
                buf.at[0], agg_sh.at[idx_v.at[0]], ssem).wait()
        plsc.subcore_barrier()
        pltpu.sync_copy(agg_sh.at[pl.ds(ss * NPER, NPER)],
                        out_hbm.at[cc, pl.ds(ss * NPER, NPER)])

    f = pl.kernel(
        body,
        out_type=jax.ShapeDtypeStruct((2, N_PAD, W128), jnp.float32),
        mesh=mesh,
        scratch_types=[
            pltpu.VMEM((NCH_W, CHUNK), jnp.int32),
            pltpu.VMEM((NSLOT_S, CHUNK, W128), jnp.float32),
            pltpu.VMEM_SHARED((N_PAD, W128), jnp.float32),
            pltpu.SemaphoreType.DMA,
            pltpu.SemaphoreType.DMA,
        ],
    )
    return f(msg, idx2d, zeros)


def _edge_stage(xe, econst, wflat, rot, ones):
    I = wflat.shape[0] // K
    O = wflat.shape[1]
    used = O + C if ones else O
    grid = E_PAD // TE
    t_np = np.zeros((W128, K * I), np.float32)
    s_np = np.zeros((C, K * I), np.float32)
    for kk in range(K):
        for ii in range(I):
            t_np[ii, kk * I + ii] = 1.0
            s_np[kk, kk * I + ii] = 1.0
    t_rep = jnp.asarray(t_np, jnp.bfloat16)
    s_rep = jnp.asarray(s_np, jnp.bfloat16)

    def kern(xe_ref, ec_ref, w_ref, t_ref, s_ref, o_ref):
        xev = xe_ref[...].astype(jnp.bfloat16)
        ecb = ec_ref[...]
        dn = (((1,), (0,)), ((), ()))
        xe_rep = jax.lax.dot_general(xev, t_ref[...], dn,
                                     preferred_element_type=jnp.float32)
        pe_rep = jax.lax.dot_general(ecb, s_ref[...], dn,
                                     preferred_element_type=jnp.float32)
        z = (xe_rep * pe_rep).astype(jnp.bfloat16)
        msg = jax.lax.dot_general(z, w_ref[...], dn,
                                  preferred_element_type=jnp.float32)
        if rot:
            ecf = ecb.astype(jnp.float32)
            c1, s1 = ecf[:, 10:11], ecf[:, 11:12]
            c2, s2 = ecf[:, 12:13], ecf[:, 13:14]
            m0 = msg[:, 0:C]
            a1, b1 = msg[:, C:2 * C], msg[:, 2 * C:3 * C]
            a2, b2 = msg[:, 3 * C:4 * C], msg[:, 4 * C:5 * C]
            msg = jnp.concatenate(
                [m0, c1 * a1 - s1 * b1, s1 * a1 + c1 * b1,
                 c2 * a2 - s2 * b2, s2 * a2 + c2 * b2], axis=1)
        parts = [msg]
        if ones:
            row = (pl.program_id(0) * TE
                   + lax.broadcasted_iota(jnp.int32, (TE, C), 0))
            parts.append((row < E).astype(jnp.float32))
        if used < W128:
            parts.append(jnp.zeros((TE, W128 - used), jnp.float32))
        o_ref[...] = jnp.concatenate(parts, axis=1)

    return pl.pallas_call(
        kern,
        grid=(grid,),
        in_specs=[
            pl.BlockSpec((TE, W128), lambda i: (i, 0)),
            pl.BlockSpec((TE, C), lambda i: (i, 0)),
            pl.BlockSpec((K * I, O), lambda i: (0, 0)),
            pl.BlockSpec((W128, K * I), lambda i: (0, 0)),
            pl.BlockSpec((C, K * I), lambda i: (0, 0)),
        ],
        out_specs=pl.BlockSpec((TE, W128), lambda i: (i, 0)),
        out_shape=jax.ShapeDtypeStruct((E_PAD, W128), jnp.float32),
    )(xe, econst, wflat, t_rep, s_rep)


def _combine_first(agg2, b):
    def kern(a_ref, b_ref, x_ref, d_ref):
        a = a_ref[0] + a_ref[1]
        deg = jnp.maximum(a[:, 80:96], 1.0)
        x = a[:, :80] / deg[:, 0:1]
        x = jnp.concatenate([x[:, :C] + b_ref[...], x[:, C:],
                             jnp.zeros((TN, W128 - 80), jnp.float32)], axis=1)
        x_ref[...] = jnp.maximum(x, 0.0)
        d_ref[...] = deg

    return pl.pallas_call(
        kern,
        grid=(N // TN,),
        in_specs=[
            pl.BlockSpec((2, TN, W128), lambda i: (0, i, 0)),
            pl.BlockSpec((1, C), lambda i: (0, 0)),
        ],
        out_specs=[
            pl.BlockSpec((TN, W128), lambda i: (i, 0)),
            pl.BlockSpec((TN, C), lambda i: (i, 0)),
        ],
        out_shape=[
            jax.ShapeDtypeStruct((N, W128), jnp.float32),
            jax.ShapeDtypeStruct((N, C), jnp.float32),
        ],
    )(agg2, b)


def _combine(agg2, deg16, b, O, xprev=None, wr=None, nres=0,
             out_dtype=jnp.float32):
    def kern(*refs):
        if nres:
            a_ref, d_ref, b_ref, xp_ref, wr_ref, x_ref = refs
        else:
            a_ref, d_ref, b_ref, x_ref = refs
        a = a_ref[0] + a_ref[1]
        x = a[:, :O] / d_ref[...][:, 0:1]
        parts = []
        for p in range(O // C):
            blk = x[:, p * C:(p + 1) * C]
            if p == 0:
                blk = blk + b_ref[...]
            if nres and p < nres:
                blk = blk + jax.lax.dot_general(
                    xp_ref[...][:, p * C:(p + 1) * C].astype(jnp.float32),
                    wr_ref[...], (((1,), (0,)), ((), ())),
                    preferred_element_type=jnp.float32)
            parts.append(blk)
        parts.append(jnp.zeros((TN, W128 - O), jnp.float32))
        x_ref[...] = jnp.maximum(
            jnp.concatenate(parts, axis=1), 0.0).astype(out_dtype)

    in_specs = [
        pl.BlockSpec((2, TN, W128), lambda i: (0, i, 0)),
        pl.BlockSpec((TN, C), lambda i: (i, 0)),
        pl.BlockSpec((1, C), lambda i: (0, 0)),
    ]
    args = [agg2, deg16, b]
    if nres:
        in_specs += [
            pl.BlockSpec((TN, W128), lambda i: (i, 0)),
            pl.BlockSpec((C, C), lambda i: (0, 0)),
        ]
        args += [xprev, wr]
    return pl.pallas_call(
        kern,
        grid=(N // TN,),
        in_specs=in_specs,
        out_specs=pl.BlockSpec((TN, W128), lambda i: (i, 0)),
        out_shape=jax.ShapeDtypeStruct((N, W128), out_dtype),
    )(*args)


def _classifier(x3, w1, b1, w2, b2):
    TB = 400

    def kern(x_ref, w1_ref, b1_ref, w2_ref, b2_ref, o_ref):
        h = jax.lax.dot_general(x_ref[...][:, :C].astype(jnp.float32),
                                w1_ref[...],
                                (((1,), (0,)), ((), ())),
                                preferred_element_type=jnp.float32)
        h = jnp.maximum(h + b1_ref[...], 0.0)
        lg = jax.lax.dot_general(h, w2_ref[...], (((1,), (0,)), ((), ())),
                                 preferred_element_type=jnp.float32)
        lg = lg + b2_ref[...]
        mx = jnp.max(lg, axis=1, keepdims=True)
        lse = jnp.log(jnp.sum(jnp.exp(lg - mx), axis=1, keepdims=True)) + mx
        o_ref[...] = lg - lse

    return pl.pallas_call(
        kern,
        grid=(N // TB,),
        in_specs=[
            pl.BlockSpec((TB, W128), lambda i: (i, 0)),
            pl.BlockSpec((C, 256), lambda i: (0, 0)),
            pl.BlockSpec((1, 256), lambda i: (0, 0)),
            pl.BlockSpec((256, NUM_CLASSES), lambda i: (0, 0)),
            pl.BlockSpec((1, NUM_CLASSES), lambda i: (0, 0)),
        ],
        out_specs=pl.BlockSpec((TB, NUM_CLASSES), lambda i: (i, 0)),
        out_shape=jax.ShapeDtypeStruct((N, NUM_CLASSES), jnp.float32),
    )(x3, w1, b1, w2, b2)


def _perm_cm(P):
    out = np.empty(P * C, np.int64)
    for p in range(P):
        for c in range(C):
            out[p * C + c] = c * P + p
    return out


def _bf(a):
    return a.astype(jnp.bfloat16)


def _prep_w(W, Pin, Pout, pad_in=None):
    if Pin is not None:
        W = W[:, _perm_cm(Pin), :]
    if Pout is not None:
        W = W[:, :, _perm_cm(Pout)]
    if pad_in is not None and pad_in > W.shape[1]:
        W = jnp.pad(W, ((0, 0), (0, pad_in - W.shape[1]), (0, 0)))
    return W.reshape(-1, W.shape[2])


def kernel(pos, edge_index, precomp, connection, Wa1, ba1, Wb1, bb1, Wr1,
           Wa2, ba2, Wb2, bb2, Wr2, Wa3, ba3, Wb3, bb3, Wr3,
           Wlin1, blin1, Wlin2, blin2):
    src = edge_index[0].astype(jnp.int32)
    dst = edge_index[1].astype(jnp.int32)
    src2d = jnp.pad(src, (0, E_PAD - E)).reshape(NJ, CHUNK)
    dst2d = jnp.pad(dst, (0, E_PAD - E),
                    constant_values=N).reshape(NJ, CHUNK)
    trig = jnp.stack([jnp.cos(connection), jnp.sin(connection),
                      jnp.cos(2.0 * connection), jnp.sin(2.0 * connection)],
                     axis=1)
    econst = jnp.pad(
        jnp.concatenate([precomp.reshape(E, K), trig], axis=1),
        ((0, E_PAD - E), (0, 2))).astype(jnp.bfloat16)

    Wa1f = _bf(_prep_w(Wa1, None, NB, pad_in=C))
    Wb1f = _bf(_prep_w(Wb1, NB, NB))
    Wa2f = _bf(_prep_w(Wa2, NB, NB))
    Wb2f = _bf(_prep_w(Wb2, NB, NB))
    Wa3f = _bf(_prep_w(Wa3, NB, None))
    Wb3f = _bf(_prep_w(Wb3, None, None))
    Wr1p = jnp.pad(Wr1, ((0, C - Wr1.shape[0]), (0, 0)))
    z128 = jnp.zeros((N_PAD, W128), jnp.float32)

    pos128 = jnp.pad(pos, ((0, 0), (0, W128 - 3)))

    xe = _sc_gather(pos128, src2d)
    msg = _edge_stage(xe, econst, Wa1f, rot=True, ones=True)
    agg = _sc_scatter(msg, dst2d, z128)
    h, deg16 = _combine_first(agg, ba1.reshape(1, C))
    xe = _sc_gather(h, src2d)
    msg = _edge_stage(xe, econst, Wb1f, rot=True, ones=False)
    agg = _sc_scatter(msg, dst2d, z128)
    x = _combine(agg, deg16, bb1.reshape(1, C), 80,
                 xprev=pos128, wr=Wr1p, nres=1)

    xe = _sc_gather(x, src2d)
    msg = _edge_stage(xe, econst, Wa2f, rot=True, ones=False)
    agg = _sc_scatter(msg, dst2d, z128)
    h = _combine(agg, deg16, ba2.reshape(1, C), 80)
    xe = _sc_gather(h, src2d)
    msg = _edge_stage(xe, econst, Wb2f, rot=True, ones=False)
    agg = _sc_scatter(msg, dst2d, z128)
    x = _combine(agg, deg16, bb2.reshape(1, C), 80, xprev=x, wr=Wr2, nres=5)

    xe = _sc_gather(x, src2d)
    msg = _edge_stage(xe, econst, Wa3f, rot=False, ones=False)
    agg = _sc_scatter(msg, dst2d, z128)
    h = _combine(agg, deg16, ba3.reshape(1, C), C)
    xe = _sc_gather(h, src2d)
    msg = _edge_stage(xe, econst, Wb3f, rot=False, ones=False)
    agg = _sc_scatter(msg, dst2d, z128)
    x = _combine(agg, deg16, bb3.reshape(1, C), C, xprev=x, wr=Wr3,
                 nres=1, out_dtype=jnp.float32)

    return _classifier(x, Wlin1, blin1.reshape(1, 256),
                       Wlin2, blin2.reshape(1, NUM_CLASSES))


# speedup vs baseline: 1.1056x; 1.1056x over previous
import functools

import numpy as np
import jax
import jax.numpy as jnp
from jax import lax
from jax.experimental import pallas as pl
from jax.experimental.pallas import tpu as pltpu
from jax.experimental.pallas import tpu_sc as plsc

N = 10000
E = 160000
NB = 5
K = 10
C = 16
NUM_CLASSES = 6890
W128 = 128

NWORK = 32
NSLOT = 6
DEPTH = 3
NSLOT_S = 2
DEPTH_S = 1
CHUNK = 128
NCH_W = 40
E_PAD = NWORK * NCH_W * CHUNK
NJ = E_PAD // CHUNK
E_HALF = E_PAD // 2
NJ_H = NJ // 2
NCH_H = NJ_H // NWORK
N_PAD = 10240
NPER = N_PAD // 16

TE = 1280
TN = 2000


def _sc_gather(table, idx3d):
    mesh = plsc.VectorSubcoreMesh(core_axis_name="c", subcore_axis_name="s")

    def body(table_hbm, idx_hbm, out_hbm, idx_v, buf, gsem, osem):
        wid = lax.axis_index("s") * 2 + lax.axis_index("c")
        base = wid * NCH_H
        pltpu.sync_copy(idx_hbm.at[wid], idx_v)
        for b in range(DEPTH):
            pltpu.async_copy(table_hbm.at[idx_v.at[b]], buf.at[b], gsem)

        def step(j, carry):
            slot = lax.rem(j, NSLOT)
            pltpu.make_async_copy(
                table_hbm.at[idx_v.at[j]], buf.at[slot], gsem).wait()
            pltpu.async_copy(buf.at[slot],
                             out_hbm.at[pl.ds((base + j) * CHUNK, CHUNK)],
                             osem)

            @pl.when(j + DEPTH < NCH_H)
            def _():
                @pl.when(j >= DEPTH)
                def _():
                    pltpu.make_async_copy(
                        buf.at[0], out_hbm.at[pl.ds(0, CHUNK)], osem).wait()
                pltpu.async_copy(
                    table_hbm.at[idx_v.at[j + DEPTH]],
                    buf.at[lax.rem(j + DEPTH, NSLOT)], gsem)

            return carry

        lax.fori_loop(0, NCH_H, step, 0)
        for _ in range(NSLOT):
            pltpu.make_async_copy(
                buf.at[0], out_hbm.at[pl.ds(0, CHUNK)], osem).wait()

    f = pl.kernel(
        body,
        out_type=jax.ShapeDtypeStruct((E_HALF, W128), jnp.float32),
        mesh=mesh,
        scratch_types=[
            pltpu.VMEM((NCH_H, CHUNK), jnp.int32),
            pltpu.VMEM((NSLOT, CHUNK, W128), jnp.float32),
            pltpu.SemaphoreType.DMA,
            pltpu.SemaphoreType.DMA,
        ],
    )
    return f(table, idx3d)


def _sc_scatter2(msgA, msgB, idx2d, zeros):
    mesh = plsc.VectorSubcoreMesh(core_axis_name="c", subcore_axis_name="s")

    def body(msgA_hbm, msgB_hbm, idx_hbm, z_hbm, out_hbm, idx_v, buf, agg_sh,
             rsem, ssem):
        cc = lax.axis_index("c")
        ss = lax.axis_index("s")
        wid = ss * 2 + cc
        localw = lax.rem(wid, 16)
        gbase = localw * NCH_W + (wid // 16) * NJ_H
        pltpu.sync_copy(z_hbm.at[pl.ds(ss * NPER, NPER)],
                        agg_sh.at[pl.ds(ss * NPER, NPER)])
        pltpu.sync_copy(idx_hbm.at[pl.ds(gbase, NCH_W)], idx_v)
        plsc.subcore_barrier()

        def ring(msg_hbm):
            lb = localw * NCH_W
            for b in range(DEPTH_S):
                pltpu.async_copy(
                    msg_hbm.at[pl.ds((lb + b) * CHUNK, CHUNK)], buf.at[b],
                    rsem)

            def step(j, carry):
                slot = lax.rem(j, NSLOT_S)
                pltpu.make_async_copy(
                    msg_hbm.at[pl.ds((lb + j) * CHUNK, CHUNK)], buf.at[slot],
                    rsem).wait()
                pltpu.async_copy(buf.at[slot], agg_sh.at[idx_v.at[j]], ssem,
                                 add=True)

                @pl.when(j + DEPTH_S < NCH_W)
                def _():
                    @pl.when(j >= DEPTH_S)
                    def _():
                        pltpu.make_async_copy(
                            buf.at[0], agg_sh.at[idx_v.at[0]], ssem).wait()
                    pltpu.async_copy(
                        msg_hbm.at[pl.ds((lb + j + DEPTH_S) * CHUNK, CHUNK)],
                        buf.at[lax.rem(j + DEPTH_S, NSLOT_S)], rsem)

                return carry

            lax.fori_loop(0, NCH_W, step, 0)
            for _ in range(NSLOT_S):
                pltpu.make_async_copy(
                    buf.at[0], agg_sh.at[idx_v.at[0]], ssem).wait()

        @pl.when(wid < 16)
        def _():
            ring(msgA_hbm)

        @pl.when(wid >= 16)
        def _():
            ring(msgB_hbm)

        plsc.subcore_barrier()
        pltpu.sync_copy(agg_sh.at[pl.ds(ss * NPER, NPER)],
                        out_hbm.at[cc, pl.ds(ss * NPER, NPER)])

    f = pl.kernel(
        body,
        out_type=jax.ShapeDtypeStruct((2, N_PAD, W128), jnp.float32),
        mesh=mesh,
        scratch_types=[
            pltpu.VMEM((NCH_W, CHUNK), jnp.int32),
            pltpu.VMEM((NSLOT_S, CHUNK, W128), jnp.float32),
            pltpu.VMEM_SHARED((N_PAD, W128), jnp.float32),
            pltpu.SemaphoreType.DMA,
            pltpu.SemaphoreType.DMA,
        ],
    )
    return f(msgA, msgB, idx2d, zeros)


def _edge_stage(xe, econst, wflat, rot, ones, half):
    I = wflat.shape[0] // K
    O = wflat.shape[1]
    used = O + C if ones else O
    grid = E_HALF // TE
    t_np = np.zeros((W128, K * I), np.float32)
    s_np = np.zeros((C, K * I), np.float32)
    for kk in range(K):
        for ii in range(I):
            t_np[ii, kk * I + ii] = 1.0
            s_np[kk, kk * I + ii] = 1.0
    t_rep = jnp.asarray(t_np, jnp.bfloat16)
    s_rep = jnp.asarray(s_np, jnp.bfloat16)

    def kern(xe_ref, ec_ref, w_ref, t_ref, s_ref, o_ref):
        xev = xe_ref[...].astype(jnp.bfloat16)
        ecb = ec_ref[...]
        dn = (((1,), (0,)), ((), ()))
        xe_rep = jax.lax.dot_general(xev, t_ref[...], dn,
                                     preferred_element_type=jnp.float32)
        pe_rep = jax.lax.dot_general(ecb, s_ref[...], dn,
                                     preferred_element_type=jnp.float32)
        z = (xe_rep * pe_rep).astype(jnp.bfloat16)
        msg = jax.lax.dot_general(z, w_ref[...], dn,
                                  preferred_element_type=jnp.float32)
        if rot:
            ecf = ecb.astype(jnp.float32)
            c1, s1 = ecf[:, 10:11], ecf[:, 11:12]
            c2, s2 = ecf[:, 12:13], ecf[:, 13:14]
            m0 = msg[:, 0:C]
            a1, b1 = msg[:, C:2 * C], msg[:, 2 * C:3 * C]
            a2, b2 = msg[:, 3 * C:4 * C], msg[:, 4 * C:5 * C]
            msg = jnp.concatenate(
                [m0, c1 * a1 - s1 * b1, s1 * a1 + c1 * b1,
                 c2 * a2 - s2 * b2, s2 * a2 + c2 * b2], axis=1)
        parts = [msg]
        if ones:
            row = (half * E_HALF + pl.program_id(0) * TE
                   + lax.broadcasted_iota(jnp.int32, (TE, C), 0))
            parts.append((row < E).astype(jnp.float32))
        if used < W128:
            parts.append(jnp.zeros((TE, W128 - used), jnp.float32))
        o_ref[...] = jnp.concatenate(parts, axis=1)

    return pl.pallas_call(
        kern,
        grid=(grid,),
        in_specs=[
            pl.BlockSpec((TE, W128), lambda i: (i, 0)),
            pl.BlockSpec((TE, C), lambda i: (i + half * (E_HALF // TE), 0)),
            pl.BlockSpec((K * I, O), lambda i: (0, 0)),
            pl.BlockSpec((W128, K * I), lambda i: (0, 0)),
            pl.BlockSpec((C, K * I), lambda i: (0, 0)),
        ],
        out_specs=pl.BlockSpec((TE, W128), lambda i: (i, 0)),
        out_shape=jax.ShapeDtypeStruct((E_HALF, W128), jnp.float32),
    )(xe, econst, wflat, t_rep, s_rep)


def _combine_first(agg2, b):
    def kern(a_ref, b_ref, x_ref, d_ref):
        a = a_ref[0] + a_ref[1]
        deg = jnp.maximum(a[:, 80:96], 1.0)
        x = a[:, :80] / deg[:, 0:1]
        x = jnp.concatenate([x[:, :C] + b_ref[...], x[:, C:],
                             jnp.zeros((TN, W128 - 80), jnp.float32)], axis=1)
        x_ref[...] = jnp.maximum(x, 0.0)
        d_ref[...] = deg

    return pl.pallas_call(
        kern,
        grid=(N // TN,),
        in_specs=[
            pl.BlockSpec((2, TN, W128), lambda i: (0, i, 0)),
            pl.BlockSpec((1, C), lambda i: (0, 0)),
        ],
        out_specs=[
            pl.BlockSpec((TN, W128), lambda i: (i, 0)),
            pl.BlockSpec((TN, C), lambda i: (i, 0)),
        ],
        out_shape=[
            jax.ShapeDtypeStruct((N, W128), jnp.float32),
            jax.ShapeDtypeStruct((N, C), jnp.float32),
        ],
    )(agg2, b)


def _combine(agg2, deg16, b, O, xprev=None, wr=None, nres=0,
             out_dtype=jnp.float32):
    def kern(*refs):
        if nres:
            a_ref, d_ref, b_ref, xp_ref, wr_ref, x_ref = refs
        else:
            a_ref, d_ref, b_ref, x_ref = refs
        a = a_ref[0] + a_ref[1]
        x = a[:, :O] / d_ref[...][:, 0:1]
        parts = []
        for p in range(O // C):
            blk = x[:, p * C:(p + 1) * C]
            if p == 0:
                blk = blk + b_ref[...]
            if nres and p < nres:
                blk = blk + jax.lax.dot_general(
                    xp_ref[...][:, p * C:(p + 1) * C].astype(jnp.float32),
                    wr_ref[...], (((1,), (0,)), ((), ())),
                    preferred_element_type=jnp.float32)
            parts.append(blk)
        parts.append(jnp.zeros((TN, W128 - O), jnp.float32))
        x_ref[...] = jnp.maximum(
            jnp.concatenate(parts, axis=1), 0.0).astype(out_dtype)

    in_specs = [
        pl.BlockSpec((2, TN, W128), lambda i: (0, i, 0)),
        pl.BlockSpec((TN, C), lambda i: (i, 0)),
        pl.BlockSpec((1, C), lambda i: (0, 0)),
    ]
    args = [agg2, deg16, b]
    if nres:
        in_specs += [
            pl.BlockSpec((TN, W128), lambda i: (i, 0)),
            pl.BlockSpec((C, C), lambda i: (0, 0)),
        ]
        args += [xprev, wr]
    return pl.pallas_call(
        kern,
        grid=(N // TN,),
        in_specs=in_specs,
        out_specs=pl.BlockSpec((TN, W128), lambda i: (i, 0)),
        out_shape=jax.ShapeDtypeStruct((N, W128), out_dtype),
    )(*args)


def _classifier(x3, w1, b1, w2, b2):
    TB = 400

    def kern(x_ref, w1_ref, b1_ref, w2_ref, b2_ref, o_ref):
        h = jax.lax.dot_general(x_ref[...][:, :C].astype(jnp.float32),
                                w1_ref[...],
                                (((1,), (0,)), ((), ())),
                                preferred_element_type=jnp.float32)
        h = jnp.maximum(h + b1_ref[...], 0.0)
        lg = jax.lax.dot_general(h, w2_ref[...], (((1,), (0,)), ((), ())),
                                 preferred_element_type=jnp.float32)
        lg = lg + b2_ref[...]
        mx = jnp.max(lg, axis=1, keepdims=True)
        lse = jnp.log(jnp.sum(jnp.exp(lg - mx), axis=1, keepdims=True)) + mx
        o_ref[...] = lg - lse

    return pl.pallas_call(
        kern,
        grid=(N // TB,),
        in_specs=[
            pl.BlockSpec((TB, W128), lambda i: (i, 0)),
            pl.BlockSpec((C, 256), lambda i: (0, 0)),
            pl.BlockSpec((1, 256), lambda i: (0, 0)),
            pl.BlockSpec((256, NUM_CLASSES), lambda i: (0, 0)),
            pl.BlockSpec((1, NUM_CLASSES), lambda i: (0, 0)),
        ],
        out_specs=pl.BlockSpec((TB, NUM_CLASSES), lambda i: (i, 0)),
        out_shape=jax.ShapeDtypeStruct((N, NUM_CLASSES), jnp.float32),
    )(x3, w1, b1, w2, b2)


def _perm_cm(P):
    out = np.empty(P * C, np.int64)
    for p in range(P):
        for c in range(C):
            out[p * C + c] = c * P + p
    return out


def _bf(a):
    return a.astype(jnp.bfloat16)


def _prep_w(W, Pin, Pout, pad_in=None):
    if Pin is not None:
        W = W[:, _perm_cm(Pin), :]
    if Pout is not None:
        W = W[:, :, _perm_cm(Pout)]
    if pad_in is not None and pad_in > W.shape[1]:
        W = jnp.pad(W, ((0, 0), (0, pad_in - W.shape[1]), (0, 0)))
    return W.reshape(-1, W.shape[2])


def kernel(pos, edge_index, precomp, connection, Wa1, ba1, Wb1, bb1, Wr1,
           Wa2, ba2, Wb2, bb2, Wr2, Wa3, ba3, Wb3, bb3, Wr3,
           Wlin1, blin1, Wlin2, blin2):
    src = edge_index[0].astype(jnp.int32)
    dst = edge_index[1].astype(jnp.int32)
    src2d = jnp.pad(src, (0, E_PAD - E)).reshape(NJ, CHUNK)
    dst2d = jnp.pad(dst, (0, E_PAD - E),
                    constant_values=N).reshape(NJ, CHUNK)
    trig = jnp.stack([jnp.cos(connection), jnp.sin(connection),
                      jnp.cos(2.0 * connection), jnp.sin(2.0 * connection)],
                     axis=1)
    econst = jnp.pad(
        jnp.concatenate([precomp.reshape(E, K), trig], axis=1),
        ((0, E_PAD - E), (0, 2))).astype(jnp.bfloat16)

    Wa1f = _bf(_prep_w(Wa1, None, NB, pad_in=C))
    Wb1f = _bf(_prep_w(Wb1, NB, NB))
    Wa2f = _bf(_prep_w(Wa2, NB, NB))
    Wb2f = _bf(_prep_w(Wb2, NB, NB))
    Wa3f = _bf(_prep_w(Wa3, NB, None))
    Wb3f = _bf(_prep_w(Wb3, None, None))
    Wr1p = jnp.pad(Wr1, ((0, C - Wr1.shape[0]), (0, 0)))
    z128 = jnp.zeros((N_PAD, W128), jnp.float32)

    pos128 = jnp.pad(pos, ((0, 0), (0, W128 - 3)))

    srcA = src2d[:NJ_H].reshape(NWORK, NCH_H, CHUNK)
    srcB = src2d[NJ_H:].reshape(NWORK, NCH_H, CHUNK)

    def conv(table, wf, rot, ones=False):
        xeA = _sc_gather(table, srcA)
        msgA = _edge_stage(xeA, econst, wf, rot, ones, 0)
        xeB = _sc_gather(table, srcB)
        msgB = _edge_stage(xeB, econst, wf, rot, ones, 1)
        return _sc_scatter2(msgA, msgB, dst2d, z128)

    agg = conv(pos128, Wa1f, rot=True, ones=True)
    h, deg16 = _combine_first(agg, ba1.reshape(1, C))
    agg = conv(h, Wb1f, rot=True)
    x = _combine(agg, deg16, bb1.reshape(1, C), 80,
                 xprev=pos128, wr=Wr1p, nres=1)

    agg = conv(x, Wa2f, rot=True)
    h = _combine(agg, deg16, ba2.reshape(1, C), 80)
    agg = conv(h, Wb2f, rot=True)
    x = _combine(agg, deg16, bb2.reshape(1, C), 80, xprev=x, wr=Wr2, nres=5)

    agg = conv(x, Wa3f, rot=False)
    h = _combine(agg, deg16, ba3.reshape(1, C), C)
    agg = conv(h, Wb3f, rot=False)
    x = _combine(agg, deg16, bb3.reshape(1, C), C, xprev=x, wr=Wr3,
                 nres=1, out_dtype=jnp.float32)

    return _classifier(x, Wlin1, blin1.reshape(1, 256),
                       Wlin2, blin2.reshape(1, NUM_CLASSES))
